# baseline (speedup 1.0000x reference)
import jax
import jax.numpy as jnp
from jax.experimental import pallas as pl
from jax.experimental.pallas import tpu as pltpu

N = 50000
E = 800000
F_IN = 4
H = 128
B = 64


def _seg_sum(x, ids, n):
    return jax.ops.segment_sum(x, ids, num_segments=n)


def _graph_norm(x, batch, nb, w, b, ms):
    cnt = _seg_sum(jnp.ones((x.shape[0],), x.dtype), batch, nb)
    cnt = jnp.maximum(cnt, 1.0)
    mean = _seg_sum(x, batch, nb) / cnt[:, None]
    out = x - mean[batch] * ms
    var = _seg_sum(out * out, batch, nb) / cnt[:, None]
    std = jnp.sqrt(var + 1e-5)
    return w * out / std[batch] + b


def _gcn_conv(x, edge_index, W, bvec):
    n = x.shape[0]
    x = x @ W.T
    loop = jnp.arange(n)
    src = jnp.concatenate([edge_index[0], loop])
    dst = jnp.concatenate([edge_index[1], loop])
    deg = _seg_sum(jnp.ones((src.shape[0],), x.dtype), dst, n)
    dis = jnp.where(deg > 0, deg ** -0.5, 0.0)
    norm = dis[src] * dis[dst]
    out = _seg_sum(x[src] * norm[:, None], dst, n)
    return out + bvec


def _set2set(x, batch, nb, Wih, Whh, bih, bhh, steps=2):
    d = x.shape[1]
    q_star = jnp.zeros((nb, 2 * d), x.dtype)
    h = jnp.zeros((nb, d), x.dtype)
    c = jnp.zeros((nb, d), x.dtype)
    for _ in range(steps):
        gates = q_star @ Wih.T + bih + h @ Whh.T + bhh
        i, f, g, o = jnp.split(gates, 4, axis=1)
        i = jax.nn.sigmoid(i)
        f = jax.nn.sigmoid(f)
        g = jnp.tanh(g)
        o = jax.nn.sigmoid(o)
        c = f * c + i * g
        h = o * jnp.tanh(c)
        q = h
        e = jnp.sum(x * q[batch], axis=-1)
        e_max = jax.ops.segment_max(e, batch, num_segments=nb)
        e_max = jnp.where(jnp.isfinite(e_max), e_max, 0.0)
        ex = jnp.exp(e - jax.lax.stop_gradient(e_max)[batch])
        den = _seg_sum(ex, batch, nb)
        a = ex / (den[batch] + 1e-16)
        r = _seg_sum(a[:, None] * x, batch, nb)
        q_star = jnp.concatenate([q, r], axis=1)
    return q_star


def _head_body(q_ref, w1_ref, b1_ref, w2_ref, b2_ref, o_ref):
    h = jnp.maximum(
        jnp.dot(q_ref[...], w1_ref[...].T, preferred_element_type=jnp.float32)
        + b1_ref[...], 0.0)
    o_ref[...] = (
        jnp.sum(h * w2_ref[...], axis=1, keepdims=True) + b2_ref[...])


def _head(q_star, lin1_W, lin1_b, lin2_W, lin2_b):
    return pl.pallas_call(
        _head_body,
        out_shape=jax.ShapeDtypeStruct((B, lin2_W.shape[0]), jnp.float32),
    )(q_star, lin1_W, lin1_b[None, :], lin2_W, lin2_b[None, :])


def kernel(x, gn0_w, gn0_b, gn0_ms, conv1_W, conv1_b, gn1_w, gn1_b, gn1_ms,
           conv2_W, conv2_b, gn2_w, gn2_b, gn2_ms,
           lstm_Wih, lstm_Whh, lstm_bih, lstm_bhh,
           lin1_W, lin1_b, lin2_W, lin2_b, edge_index, batch):
    zero_batch = jnp.zeros((x.shape[0],), dtype=jnp.int32)
    h = _graph_norm(x, zero_batch, 1, gn0_w, gn0_b, gn0_ms)
    h = _gcn_conv(h, edge_index, conv1_W, conv1_b)
    h = jax.nn.relu(_graph_norm(h, batch, B, gn1_w, gn1_b, gn1_ms))
    h = _gcn_conv(h, edge_index, conv2_W, conv2_b)
    h = _graph_norm(h, batch, B, gn2_w, gn2_b, gn2_ms)
    h = _set2set(h, batch, B, lstm_Wih, lstm_Whh, lstm_bih, lstm_bhh, steps=2)
    return _head(h, lin1_W, lin1_b, lin2_W, lin2_b)


# speedup vs baseline: 8.6138x; 8.6138x over previous
import functools

import jax
import jax.numpy as jnp
from jax import lax
from jax.experimental import pallas as pl
from jax.experimental.pallas import tpu as pltpu
from jax.experimental.pallas import tpu_sc as plsc

N = 50000
E = 800000
F_IN = 4
H = 128
B = 64

NC = 2
NS = 16
WIN = 128
WPT = 200
EPT = WPT * WIN
E_PAD = NC * NS * EPT
EROWS = E_PAD // WIN
N_PAD = 50176
RPS = N_PAD // NS
ZCH = 448
CHW = 8

BN = 1000
GRID = N // BN

_MESH = plsc.VectorSubcoreMesh(core_axis_name="c", subcore_axis_name="s")


def _sc_deg(dst2d, ones_rows, zrows):
    @functools.partial(
        pl.kernel, mesh=_MESH,
        out_type=jax.ShapeDtypeStruct((NC, N_PAD, 16), jnp.float32),
        scratch_types=[
            pltpu.VMEM((CHW, WIN), jnp.int32),
            pltpu.VMEM((WIN, 16), jnp.float32),
            pltpu.VMEM((ZCH, 16), jnp.float32),
            pltpu.VMEM_SHARED((N_PAD, 16), jnp.float32),
        ],
        compiler_params=pltpu.CompilerParams(use_tc_tiling_on_sc=False),
    )
    def k(dst_hbm, ones_hbm, z_hbm, out, dst_v, ones_v, z_v, acc):
        c = lax.axis_index("c")
        s = lax.axis_index("s")
        pltpu.sync_copy(z_hbm, z_v)
        def zbody(j, carry):
            pltpu.sync_copy(z_v, acc.at[pl.ds(s * RPS + j * ZCH, ZCH)])
            return carry
        lax.fori_loop(0, RPS // ZCH, zbody, 0)
        pltpu.sync_copy(ones_hbm, ones_v)
        row0 = (c * NS + s) * WPT
        plsc.subcore_barrier()
        def chunk(g, carry):
            pltpu.sync_copy(dst_hbm.at[pl.ds(row0 + g * CHW, CHW)], dst_v)
            def body(w, carry2):
                pltpu.sync_copy(ones_v, acc.at[dst_v.at[w]], add=True)
                return carry2
            return lax.fori_loop(0, CHW, body, carry)
        lax.fori_loop(0, WPT // CHW, chunk, 0)
        plsc.subcore_barrier()
        pltpu.sync_copy(acc.at[pl.ds(s * RPS, RPS)],
                        out.at[c, pl.ds(s * RPS, RPS)])
    return k(dst2d, ones_rows, zrows)


def _sc_edge_agg(ncols, hp, src2d, dst2d, zrows):
    @functools.partial(
        pl.kernel, mesh=_MESH,
        out_type=jax.ShapeDtypeStruct((NC, N_PAD, ncols), jnp.float32),
        scratch_types=[
            pltpu.VMEM((CHW, WIN), jnp.int32),
            pltpu.VMEM((CHW, WIN), jnp.int32),
            pltpu.VMEM((WIN, ncols), jnp.float32),
            pltpu.VMEM((ZCH, ncols), jnp.float32),
            pltpu.VMEM_SHARED((N_PAD, ncols), jnp.float32),
        ],
        compiler_params=pltpu.CompilerParams(use_tc_tiling_on_sc=False),
    )
    def k(hp_hbm, src_hbm, dst_hbm, z_hbm, out, src_v, dst_v, rows_v, z_v,
          acc):
        c = lax.axis_index("c")
        s = lax.axis_index("s")
        pltpu.sync_copy(z_hbm, z_v)
        def zbody(j, carry):
            pltpu.sync_copy(z_v, acc.at[pl.ds(s * RPS + j * ZCH, ZCH)])
            return carry
        lax.fori_loop(0, RPS // ZCH, zbody, 0)
        row0 = (c * NS + s) * WPT
        plsc.subcore_barrier()
        def chunk(g, carry):
            pltpu.sync_copy(src_hbm.at[pl.ds(row0 + g * CHW, CHW)], src_v)
            pltpu.sync_copy(dst_hbm.at[pl.ds(row0 + g * CHW, CHW)], dst_v)
            def body(w, carry2):
                pltpu.sync_copy(hp_hbm.at[src_v.at[w]], rows_v)
                pltpu.sync_copy(rows_v, acc.at[dst_v.at[w]], add=True)
                return carry2
            return lax.fori_loop(0, CHW, body, carry)
        lax.fori_loop(0, WPT // CHW, chunk, 0)
        plsc.subcore_barrier()
        pltpu.sync_copy(acc.at[pl.ds(s * RPS, RPS)],
                        out.at[c, pl.ds(s * RPS, RPS)])
    return k(hp, src2d, dst2d, zrows)


def _onehot(batch_blk):
    iota = lax.broadcasted_iota(jnp.int32, (1, B), 1)
    return (batch_blk == iota).astype(jnp.float32)


def _bvec(onehot, table_row):
    return jnp.sum(onehot * table_row, axis=1, keepdims=True)


def _gn0_stats(x):
    def body(x_ref, s1_ref, s2_ref):
        i = pl.program_id(0)
        @pl.when(i == 0)
        def _():
            s1_ref[...] = jnp.zeros_like(s1_ref)
            s2_ref[...] = jnp.zeros_like(s2_ref)
        xb = x_ref[...]
        s1_ref[...] += jnp.sum(xb, axis=0, keepdims=True)
        s2_ref[...] += jnp.sum(xb * xb, axis=0, keepdims=True)
    return pl.pallas_call(
        body,
        grid=(GRID,),
        in_specs=[pl.BlockSpec((BN, F_IN), lambda i: (i, 0))],
        out_specs=[pl.BlockSpec((1, F_IN), lambda i: (0, 0)),
                   pl.BlockSpec((1, F_IN), lambda i: (0, 0))],
        out_shape=[jax.ShapeDtypeStruct((1, F_IN), jnp.float32)] * 2,
    )(x)


def _prep(x, s1, s2, dcop, batch2d, w, b, ms):
    def body(x_ref, s1_ref, s2_ref, d_ref, bt_ref, w_ref, b_ref, ms_ref,
             hp4_ref, xn_ref, dis_ref, cnt_ref):
        i = pl.program_id(0)
        mean = s1_ref[...] / float(N)
        msv = ms_ref[...]
        var = (s2_ref[...] / float(N)
               - (2.0 * msv - msv * msv) * mean * mean)
        std = jnp.sqrt(var + 1e-5)
        xb = x_ref[...]
        xn = w_ref[...] * (xb - mean * msv) / std + b_ref[...]
        d = d_ref[...]
        deg = d[0, :, 0:1] + d[1, :, 0:1] + 1.0
        dis = lax.rsqrt(deg)
        xn_ref[...] = xn
        dis_ref[...] = jnp.broadcast_to(dis, dis_ref.shape)
        hp4_ref[...] = jnp.concatenate(
            [xn * dis, jnp.zeros((xn.shape[0], 16 - F_IN), jnp.float32)],
            axis=1)
        oh = _onehot(bt_ref[...])
        @pl.when(i == 0)
        def _():
            cnt_ref[...] = jnp.zeros_like(cnt_ref)
        cnt_ref[...] += jnp.sum(oh, axis=0, keepdims=True)
    return pl.pallas_call(
        body,
        grid=(GRID,),
        in_specs=[
            pl.BlockSpec((BN, F_IN), lambda i: (i, 0)),
            pl.BlockSpec((1, F_IN), lambda i: (0, 0)),
            pl.BlockSpec((1, F_IN), lambda i: (0, 0)),
            pl.BlockSpec((NC, BN, 16), lambda i: (0, i, 0)),
            pl.BlockSpec((BN, 1), lambda i: (i, 0)),
            pl.BlockSpec((1, F_IN), lambda i: (0, 0)),
            pl.BlockSpec((1, F_IN), lambda i: (0, 0)),
            pl.BlockSpec((1, F_IN), lambda i: (0, 0)),
        ],
        out_specs=[
            pl.BlockSpec((BN, 16), lambda i: (i, 0)),
            pl.BlockSpec((BN, F_IN), lambda i: (i, 0)),
            pl.BlockSpec((BN, 8), lambda i: (i, 0)),
            pl.BlockSpec((1, B), lambda i: (0, 0)),
        ],
        out_shape=[
            jax.ShapeDtypeStruct((N, 16), jnp.float32),
            jax.ShapeDtypeStruct((N, F_IN), jnp.float32),
            jax.ShapeDtypeStruct((N, 8), jnp.float32),
            jax.ShapeDtypeStruct((1, B), jnp.float32),
        ],
    )(x, s1, s2, dcop, batch2d, w[None, :], b[None, :], ms[None, :])


def _conv1(a4, xn, dis8, batch2d, W1T, b1):
    def body(a_ref, xn_ref, dis_ref, bt_ref, w_ref, b_ref,
             h1_ref, s1_ref, s2_ref):
        i = pl.program_id(0)
        a = a_ref[...]
        dis = dis_ref[...][:, 0:1]
        agg = (a[0, :, 0:F_IN] + a[1, :, 0:F_IN])
        A4 = dis * agg + (dis * dis) * xn_ref[...]
        h1 = jnp.dot(A4, w_ref[...],
                     preferred_element_type=jnp.float32) + b_ref[...]
        h1_ref[...] = h1
        oh = _onehot(bt_ref[...])
        @pl.when(i == 0)
        def _():
            s1_ref[...] = jnp.zeros_like(s1_ref)
            s2_ref[...] = jnp.zeros_like(s2_ref)
        s1_ref[...] += lax.dot_general(oh, h1, (((0,), (0,)), ((), ())),
                                       precision=lax.Precision.HIGHEST,
                                       preferred_element_type=jnp.float32)
        s2_ref[...] += lax.dot_general(oh, h1 * h1, (((0,), (0,)), ((), ())),
                                       precision=lax.Precision.HIGHEST,
                                       preferred_element_type=jnp.float32)
    return pl.pallas_call(
        body,
        grid=(GRID,),
        in_specs=[
            pl.BlockSpec((NC, BN, 16), lambda i: (0, i, 0)),
            pl.BlockSpec((BN, F_IN), lambda i: (i, 0)),
            pl.BlockSpec((BN, 8), lambda i: (i, 0)),
            pl.BlockSpec((BN, 1), lambda i: (i, 0)),
            pl.BlockSpec((F_IN, H), lambda i: (0, 0)),
            pl.BlockSpec((1, H), lambda i: (0, 0)),
        ],
        out_specs=[
            pl.BlockSpec((BN, H), lambda i: (i, 0)),
            pl.BlockSpec((B, H), lambda i: (0, 0)),
            pl.BlockSpec((B, H), lambda i: (0, 0)),
        ],
        out_shape=[
            jax.ShapeDtypeStruct((N, H), jnp.float32),
            jax.ShapeDtypeStruct((B, H), jnp.float32),
            jax.ShapeDtypeStruct((B, H), jnp.float32),
        ],
    )(a4, xn, dis8, batch2d, W1T, b1[None, :])


def _gn_apply(h, batch2d, s1, s2, cnt, w, b, ms, dis8=None, relu=False):
    split = dis8 is not None

    def body(*refs):
        if split:
            (h_ref, bt_ref, s1_ref, s2_ref, cnt_ref, w_ref, b_ref, ms_ref,
             dis_ref, o0, o1, o2, o3) = refs
        else:
            (h_ref, bt_ref, s1_ref, s2_ref, cnt_ref, w_ref, b_ref, ms_ref,
             og) = refs
        oh = _onehot(bt_ref[...])
        cntn = jnp.maximum(_bvec(oh, cnt_ref[...]), 1.0)
        s1n = jnp.dot(oh, s1_ref[...], precision=lax.Precision.HIGHEST,
                      preferred_element_type=jnp.float32)
        s2n = jnp.dot(oh, s2_ref[...], precision=lax.Precision.HIGHEST,
                      preferred_element_type=jnp.float32)
        mean = s1n / cntn
        msv = ms_ref[...]
        var = s2n / cntn - (2.0 * msv - msv * msv) * mean * mean
        std = jnp.sqrt(var + 1e-5)
        out = w_ref[...] * (h_ref[...] - mean * msv) / std + b_ref[...]
        if relu:
            out = jnp.maximum(out, 0.0)
        if split:
            out = out * dis_ref[...][:, 0:1]
            o0[...] = out[:, 0:32]
            o1[...] = out[:, 32:64]
            o2[...] = out[:, 64:96]
            o3[...] = out[:, 96:128]
        else:
            og[...] = out

    in_specs = [
        pl.BlockSpec((BN, H), lambda i: (i, 0)),
        pl.BlockSpec((BN, 1), lambda i: (i, 0)),
        pl.BlockSpec((B, H), lambda i: (0, 0)),
        pl.BlockSpec((B, H), lambda i: (0, 0)),
        pl.BlockSpec((1, B), lambda i: (0, 0)),
        pl.BlockSpec((1, H), lambda i: (0, 0)),
        pl.BlockSpec((1, H), lambda i: (0, 0)),
        pl.BlockSpec((1, H), lambda i: (0, 0)),
    ]
    args = [h, batch2d, s1, s2, cnt, w[None, :], b[None, :], ms[None, :]]
    if split:
        in_specs.append(pl.BlockSpec((BN, 8), lambda i: (i, 0)))
        args.append(dis8)
        out_specs = [pl.BlockSpec((BN, 32), lambda i: (i, 0))] * 4
        out_shape = [jax.ShapeDtypeStruct((N, 32), jnp.float32)] * 4
    else:
        out_specs = [pl.BlockSpec((BN, H), lambda i: (i, 0))]
        out_shape = [jax.ShapeDtypeStruct((N, H), jnp.float32)]
    return pl.pallas_call(
        body, grid=(GRID,), in_specs=in_specs,
        out_specs=out_specs, out_shape=out_shape,
    )(*args)


def _conv2(aggs, hps, dis8, batch2d, W2T, b2):
    def body(a0, a1, a2, a3, p0, p1, p2, p3, dis_ref, bt_ref, w_ref, b_ref,
             h3_ref, s1_ref, s2_ref):
        i = pl.program_id(0)
        dis = dis_ref[...][:, 0:1]
        parts = []
        for a_ref, p_ref in ((a0, p0), (a1, p1), (a2, p2), (a3, p3)):
            a = a_ref[...]
            parts.append(dis * (a[0] + a[1] + p_ref[...]))
        A = jnp.concatenate(parts, axis=1)
        h3 = jnp.dot(A, w_ref[...],
                     preferred_element_type=jnp.float32) + b_ref[...]
        h3_ref[...] = h3
        oh = _onehot(bt_ref[...])
        @pl.when(i == 0)
        def _():
            s1_ref[...] = jnp.zeros_like(s1_ref)
            s2_ref[...] = jnp.zeros_like(s2_ref)
        s1_ref[...] += lax.dot_general(oh, h3, (((0,), (0,)), ((), ())),
                                       precision=lax.Precision.HIGHEST,
                                       preferred_element_type=jnp.float32)
        s2_ref[...] += lax.dot_general(oh, h3 * h3, (((0,), (0,)), ((), ())),
                                       precision=lax.Precision.HIGHEST,
                                       preferred_element_type=jnp.float32)
    return pl.pallas_call(
        body,
        grid=(GRID,),
        in_specs=(
            [pl.BlockSpec((NC, BN, 32), lambda i: (0, i, 0))] * 4
            + [pl.BlockSpec((BN, 32), lambda i: (i, 0))] * 4
            + [pl.BlockSpec((BN, 8), lambda i: (i, 0)),
               pl.BlockSpec((BN, 1), lambda i: (i, 0)),
               pl.BlockSpec((H, H), lambda i: (0, 0)),
               pl.BlockSpec((1, H), lambda i: (0, 0))]),
        out_specs=[
            pl.BlockSpec((BN, H), lambda i: (i, 0)),
            pl.BlockSpec((B, H), lambda i: (0, 0)),
            pl.BlockSpec((B, H), lambda i: (0, 0)),
        ],
        out_shape=[
            jax.ShapeDtypeStruct((N, H), jnp.float32),
            jax.ShapeDtypeStruct((B, H), jnp.float32),
            jax.ShapeDtypeStruct((B, H), jnp.float32),
        ],
    )(*aggs, *hps, dis8, batch2d, W2T, b2[None, :])


def _lstm_consts(bih, bhh):
    g = bih + bhh
    i = jax.nn.sigmoid(g[:, 0:H])
    f = jax.nn.sigmoid(g[:, H:2 * H])
    gg = jnp.tanh(g[:, 2 * H:3 * H])
    o = jax.nn.sigmoid(g[:, 3 * H:4 * H])
    c1 = f * 0.0 + i * gg
    q1 = o * jnp.tanh(c1)
    return q1, c1


def _e_pass(hg, batch2d, q, is_table):
    def body(hg_ref, bt_ref, q_ref, e_ref, m_ref):
        i = pl.program_id(0)
        if is_table:
            oh = _onehot(bt_ref[...])
            qn = jnp.dot(oh, q_ref[...], precision=lax.Precision.HIGHEST,
                         preferred_element_type=jnp.float32)
        else:
            qn = q_ref[...]
        e = jnp.sum(hg_ref[...] * qn, axis=1, keepdims=True)
        e_ref[...] = jnp.broadcast_to(e, e_ref.shape)
        bm = jnp.max(e)
        @pl.when(i == 0)
        def _():
            m_ref[...] = jnp.full_like(m_ref, -jnp.inf)
        m_ref[...] = jnp.maximum(m_ref[...], bm)
    return pl.pallas_call(
        body,
        grid=(GRID,),
        in_specs=[
            pl.BlockSpec((BN, H), lambda i: (i, 0)),
            pl.BlockSpec((BN, 1), lambda i: (i, 0)),
            pl.BlockSpec((B if is_table else 1, H), lambda i: (0, 0)),
        ],
        out_specs=[
            pl.BlockSpec((BN, 8), lambda i: (i, 0)),
            pl.BlockSpec((1, 8), lambda i: (0, 0)),
        ],
        out_shape=[
            jax.ShapeDtypeStruct((N, 8), jnp.float32),
            jax.ShapeDtypeStruct((1, 8), jnp.float32),
        ],
    )(hg, batch2d, q)


def _den_pass(e8, m8, batch2d):
    def body(e_ref, m_ref, bt_ref, ex_ref, den_ref):
        i = pl.program_id(0)
        ex = jnp.exp(e_ref[...][:, 0:1] - m_ref[...][0:1, 0:1])
        ex_ref[...] = jnp.broadcast_to(ex, ex_ref.shape)
        oh = _onehot(bt_ref[...])
        @pl.when(i == 0)
        def _():
            den_ref[...] = jnp.zeros_like(den_ref)
        den_ref[...] += jnp.sum(oh * ex, axis=0, keepdims=True)
    return pl.pallas_call(
        body,
        grid=(GRID,),
        in_specs=[
            pl.BlockSpec((BN, 8), lambda i: (i, 0)),
            pl.BlockSpec((1, 8), lambda i: (0, 0)),
            pl.BlockSpec((BN, 1), lambda i: (i, 0)),
        ],
        out_specs=[
            pl.BlockSpec((BN, 8), lambda i: (i, 0)),
            pl.BlockSpec((1, B), lambda i: (0, 0)),
        ],
        out_shape=[
            jax.ShapeDtypeStruct((N, 8), jnp.float32),
            jax.ShapeDtypeStruct((1, B), jnp.float32),
        ],
    )(e8, m8, batch2d)


def _r_pass(hg, ex8, den, batch2d):
    def body(hg_ref, ex_ref, den_ref, bt_ref, r_ref):
        i = pl.program_id(0)
        oh = _onehot(bt_ref[...])
        dn = _bvec(oh, den_ref[...]) + 1e-16
        a = ex_ref[...][:, 0:1] / dn
        @pl.when(i == 0)
        def _():
            r_ref[...] = jnp.zeros_like(r_ref)
        r_ref[...] += lax.dot_general(oh, hg_ref[...] * a,
                                      (((0,), (0,)), ((), ())),
                                      precision=lax.Precision.HIGHEST,
                                      preferred_element_type=jnp.float32)
    return pl.pallas_call(
        body,
        grid=(GRID,),
        in_specs=[
            pl.BlockSpec((BN, H), lambda i: (i, 0)),
            pl.BlockSpec((BN, 8), lambda i: (i, 0)),
            pl.BlockSpec((1, B), lambda i: (0, 0)),
            pl.BlockSpec((BN, 1), lambda i: (i, 0)),
        ],
        out_specs=[pl.BlockSpec((B, H), lambda i: (0, 0))],
        out_shape=[jax.ShapeDtypeStruct((B, H), jnp.float32)],
    )(hg, ex8, den, batch2d)


def _lstm2(r1, WihT, WhhT, bih, bhh):
    def body(r_ref, wih_ref, whh_ref, bih_ref, bhh_ref, q2_ref):
        q1, c1 = _lstm_consts(bih_ref[...], bhh_ref[...])
        q1b = jnp.broadcast_to(q1, (B, H))
        qs1 = jnp.concatenate([q1b, r_ref[...]], axis=1)
        gates = (jnp.dot(qs1, wih_ref[...],
                         preferred_element_type=jnp.float32) + bih_ref[...]
                 + jnp.dot(q1b, whh_ref[...],
                           preferred_element_type=jnp.float32) + bhh_ref[...])
        i = jax.nn.sigmoid(gates[:, 0:H])
        f = jax.nn.sigmoid(gates[:, H:2 * H])
        g = jnp.tanh(gates[:, 2 * H:3 * H])
        o = jax.nn.sigmoid(gates[:, 3 * H:4 * H])
        c2 = f * c1 + i * g
        q2_ref[...] = o * jnp.tanh(c2)
    return pl.pallas_call(
        body,
        out_shape=jax.ShapeDtypeStruct((B, H), jnp.float32),
    )(r1, WihT, WhhT, bih, bhh)


def _head(q2, r2, lin1T, b1, lin2_W, b2):
    def body(q_ref, r_ref, w1_ref, b1_ref, w2_ref, b2_ref, o_ref):
        qs = jnp.concatenate([q_ref[...], r_ref[...]], axis=1)
        h = jnp.maximum(
            jnp.dot(qs, w1_ref[...], preferred_element_type=jnp.float32)
            + b1_ref[...], 0.0)
        o_ref[...] = (jnp.sum(h * w2_ref[...], axis=1, keepdims=True)
                      + b2_ref[...])
    return pl.pallas_call(
        body,
        out_shape=jax.ShapeDtypeStruct((B, 1), jnp.float32),
    )(q2, r2, lin1T, b1[None, :], lin2_W, b2[None, :])


def kernel(x, gn0_w, gn0_b, gn0_ms, conv1_W, conv1_b, gn1_w, gn1_b, gn1_ms,
           conv2_W, conv2_b, gn2_w, gn2_b, gn2_ms,
           lstm_Wih, lstm_Whh, lstm_bih, lstm_bhh,
           lin1_W, lin1_b, lin2_W, lin2_b, edge_index, batch):
    npad = E_PAD - E
    src2d = jnp.concatenate(
        [edge_index[0], jnp.zeros((npad,), jnp.int32)]).reshape(EROWS, WIN)
    dst2d = jnp.concatenate(
        [edge_index[1],
         N + (jnp.arange(npad, dtype=jnp.int32) % WIN)]).reshape(EROWS, WIN)
    batch2d = batch.astype(jnp.int32).reshape(N, 1)
    zrows16 = jnp.zeros((ZCH, 16), jnp.float32)
    zrows32 = jnp.zeros((ZCH, 32), jnp.float32)
    ones_rows = jnp.concatenate(
        [jnp.ones((WIN, 1), jnp.float32), jnp.zeros((WIN, 15), jnp.float32)],
        axis=1)

    dcop = _sc_deg(dst2d, ones_rows, zrows16)
    s1x, s2x = _gn0_stats(x)
    hp4, xn, dis8, cnt = _prep(x, s1x, s2x, dcop, batch2d, gn0_w, gn0_b,
                               gn0_ms)
    a4 = _sc_edge_agg(16, hp4, src2d, dst2d, zrows16)
    h1, s1a, s2a = _conv1(a4, xn, dis8, batch2d, conv1_W.T, conv1_b)
    hps = _gn_apply(h1, batch2d, s1a, s2a, cnt, gn1_w, gn1_b, gn1_ms,
                    dis8=dis8, relu=True)
    aggs = [_sc_edge_agg(32, hp_c, src2d, dst2d, zrows32) for hp_c in hps]
    h3, s1b, s2b = _conv2(aggs, hps, dis8, batch2d, conv2_W.T, conv2_b)
    (hg,) = _gn_apply(h3, batch2d, s1b, s2b, cnt, gn2_w, gn2_b, gn2_ms)

    bih2 = lstm_bih[None, :]
    bhh2 = lstm_bhh[None, :]
    q1, _ = _lstm_consts(bih2, bhh2)
    e1, m1 = _e_pass(hg, batch2d, q1, is_table=False)
    ex1, den1 = _den_pass(e1, m1, batch2d)
    r1 = _r_pass(hg, ex1, den1, batch2d)[0]
    q2 = _lstm2(r1, lstm_Wih.T, lstm_Whh.T, bih2, bhh2)
    e2, m2 = _e_pass(hg, batch2d, q2, is_table=True)
    ex2, den2 = _den_pass(e2, m2, batch2d)
    r2 = _r_pass(hg, ex2, den2, batch2d)[0]
    return _head(q2, r2, lin1_W.T, lin1_b, lin2_W, lin2_b)


# speedup vs baseline: 9.2904x; 1.0786x over previous
import functools

import jax
import jax.numpy as jnp
from jax import lax
from jax.experimental import pallas as pl
from jax.experimental.pallas import tpu as pltpu
from jax.experimental.pallas import tpu_sc as plsc

N = 50000
E = 800000
F_IN = 4
H = 128
B = 64

NC = 2
NS = 16
WIN = 128
WPT = 200
EPT = WPT * WIN
E_PAD = NC * NS * EPT
EROWS = E_PAD // WIN
N_PAD = 50176
RPS = N_PAD // NS
ZCH = 112
CHW = 4

BN = 1000
GRID = N // BN

_MESH = plsc.VectorSubcoreMesh(core_axis_name="c", subcore_axis_name="s")


def _sc_deg(dst2d, ones_rows, zrows):
    @functools.partial(
        pl.kernel, mesh=_MESH,
        out_type=jax.ShapeDtypeStruct((NC, N_PAD, 16), jnp.float32),
        scratch_types=[
            pltpu.VMEM((CHW * WIN,), jnp.int32),
            pltpu.VMEM((CHW * WIN, 16), jnp.float32),
            pltpu.VMEM((ZCH, 16), jnp.float32),
            pltpu.VMEM_SHARED((N_PAD, 16), jnp.float32),
        ],
        compiler_params=pltpu.CompilerParams(use_tc_tiling_on_sc=False),
    )
    def k(dst_hbm, ones_hbm, z_hbm, out, dst_v, ones_v, z_v, acc):
        c = lax.axis_index("c")
        s = lax.axis_index("s")
        pltpu.sync_copy(z_hbm, z_v)
        def zbody(j, carry):
            pltpu.sync_copy(z_v, acc.at[pl.ds(s * RPS + j * ZCH, ZCH)])
            return carry
        lax.fori_loop(0, RPS // ZCH, zbody, 0)
        pltpu.sync_copy(ones_hbm, ones_v)
        e0 = (c * NS + s) * EPT
        plsc.subcore_barrier()
        def chunk(g, carry):
            pltpu.sync_copy(dst_hbm.at[pl.ds(e0 + g * (CHW * WIN),
                                             CHW * WIN)], dst_v)
            pltpu.sync_copy(ones_v, acc.at[dst_v], add=True)
            return carry
        lax.fori_loop(0, WPT // CHW, chunk, 0)
        plsc.subcore_barrier()
        pltpu.sync_copy(acc.at[pl.ds(s * RPS, RPS)],
                        out.at[c, pl.ds(s * RPS, RPS)])
    return k(dst2d, ones_rows, zrows)


def _sc_edge_agg(ncols, hp, src2d, dst2d, zrows):
    @functools.partial(
        pl.kernel, mesh=_MESH,
        out_type=jax.ShapeDtypeStruct((NC, N_PAD, ncols), jnp.float32),
        scratch_types=[
            pltpu.VMEM((CHW * WIN,), jnp.int32),
            pltpu.VMEM((CHW * WIN,), jnp.int32),
            pltpu.VMEM((CHW * WIN, ncols), jnp.float32),
            pltpu.VMEM((ZCH, ncols), jnp.float32),
            pltpu.VMEM_SHARED((N_PAD, ncols), jnp.float32),
        ],
        compiler_params=pltpu.CompilerParams(use_tc_tiling_on_sc=False),
    )
    def k(hp_hbm, src_hbm, dst_hbm, z_hbm, out, src_v, dst_v, rows_v, z_v,
          acc):
        c = lax.axis_index("c")
        s = lax.axis_index("s")
        pltpu.sync_copy(z_hbm, z_v)
        def zbody(j, carry):
            pltpu.sync_copy(z_v, acc.at[pl.ds(s * RPS + j * ZCH, ZCH)])
            return carry
        lax.fori_loop(0, RPS // ZCH, zbody, 0)
        e0 = (c * NS + s) * EPT
        plsc.subcore_barrier()
        def chunk(g, carry):
            pltpu.sync_copy(src_hbm.at[pl.ds(e0 + g * (CHW * WIN),
                                             CHW * WIN)], src_v)
            pltpu.sync_copy(dst_hbm.at[pl.ds(e0 + g * (CHW * WIN),
                                             CHW * WIN)], dst_v)
            pltpu.sync_copy(hp_hbm.at[src_v], rows_v)
            pltpu.sync_copy(rows_v, acc.at[dst_v], add=True)
            return carry
        lax.fori_loop(0, WPT // CHW, chunk, 0)
        plsc.subcore_barrier()
        pltpu.sync_copy(acc.at[pl.ds(s * RPS, RPS)],
                        out.at[c, pl.ds(s * RPS, RPS)])
    return k(hp, src2d, dst2d, zrows)


def _onehot(batch_blk):
    iota = lax.broadcasted_iota(jnp.int32, (1, B), 1)
    return (batch_blk == iota).astype(jnp.float32)


def _bvec(onehot, table_row):
    return jnp.sum(onehot * table_row, axis=1, keepdims=True)


def _gn0_stats(x):
    def body(x_ref, s1_ref, s2_ref):
        i = pl.program_id(0)
        @pl.when(i == 0)
        def _():
            s1_ref[...] = jnp.zeros_like(s1_ref)
            s2_ref[...] = jnp.zeros_like(s2_ref)
        xb = x_ref[...]
        s1_ref[...] += jnp.sum(xb, axis=0, keepdims=True)
        s2_ref[...] += jnp.sum(xb * xb, axis=0, keepdims=True)
    return pl.pallas_call(
        body,
        grid=(GRID,),
        in_specs=[pl.BlockSpec((BN, F_IN), lambda i: (i, 0))],
        out_specs=[pl.BlockSpec((1, F_IN), lambda i: (0, 0)),
                   pl.BlockSpec((1, F_IN), lambda i: (0, 0))],
        out_shape=[jax.ShapeDtypeStruct((1, F_IN), jnp.float32)] * 2,
    )(x)


def _prep(x, s1, s2, dcop, batch2d, w, b, ms):
    def body(x_ref, s1_ref, s2_ref, d_ref, bt_ref, w_ref, b_ref, ms_ref,
             hp4_ref, xn_ref, dis_ref, cnt_ref):
        i = pl.program_id(0)
        mean = s1_ref[...] / float(N)
        msv = ms_ref[...]
        var = (s2_ref[...] / float(N)
               - (2.0 * msv - msv * msv) * mean * mean)
        std = jnp.sqrt(var + 1e-5)
        xb = x_ref[...]
        xn = w_ref[...] * (xb - mean * msv) / std + b_ref[...]
        d = d_ref[...]
        deg = d[0, :, 0:1] + d[1, :, 0:1] + 1.0
        dis = lax.rsqrt(deg)
        xn_ref[...] = xn
        dis_ref[...] = jnp.broadcast_to(dis, dis_ref.shape)
        hp4_ref[...] = jnp.concatenate(
            [xn * dis, jnp.zeros((xn.shape[0], 16 - F_IN), jnp.float32)],
            axis=1)
        oh = _onehot(bt_ref[...])
        @pl.when(i == 0)
        def _():
            cnt_ref[...] = jnp.zeros_like(cnt_ref)
        cnt_ref[...] += jnp.sum(oh, axis=0, keepdims=True)
    return pl.pallas_call(
        body,
        grid=(GRID,),
        in_specs=[
            pl.BlockSpec((BN, F_IN), lambda i: (i, 0)),
            pl.BlockSpec((1, F_IN), lambda i: (0, 0)),
            pl.BlockSpec((1, F_IN), lambda i: (0, 0)),
            pl.BlockSpec((NC, BN, 16), lambda i: (0, i, 0)),
            pl.BlockSpec((BN, 1), lambda i: (i, 0)),
            pl.BlockSpec((1, F_IN), lambda i: (0, 0)),
            pl.BlockSpec((1, F_IN), lambda i: (0, 0)),
            pl.BlockSpec((1, F_IN), lambda i: (0, 0)),
        ],
        out_specs=[
            pl.BlockSpec((BN, 16), lambda i: (i, 0)),
            pl.BlockSpec((BN, F_IN), lambda i: (i, 0)),
            pl.BlockSpec((BN, 8), lambda i: (i, 0)),
            pl.BlockSpec((1, B), lambda i: (0, 0)),
        ],
        out_shape=[
            jax.ShapeDtypeStruct((N, 16), jnp.float32),
            jax.ShapeDtypeStruct((N, F_IN), jnp.float32),
            jax.ShapeDtypeStruct((N, 8), jnp.float32),
            jax.ShapeDtypeStruct((1, B), jnp.float32),
        ],
    )(x, s1, s2, dcop, batch2d, w[None, :], b[None, :], ms[None, :])


def _conv1(a4, xn, dis8, batch2d, W1T, b1):
    def body(a_ref, xn_ref, dis_ref, bt_ref, w_ref, b_ref,
             h1_ref, s1_ref, s2_ref):
        i = pl.program_id(0)
        a = a_ref[...]
        dis = dis_ref[...][:, 0:1]
        agg = (a[0, :, 0:F_IN] + a[1, :, 0:F_IN])
        A4 = dis * agg + (dis * dis) * xn_ref[...]
        h1 = jnp.dot(A4, w_ref[...],
                     preferred_element_type=jnp.float32) + b_ref[...]
        h1_ref[...] = h1
        oh = _onehot(bt_ref[...])
        @pl.when(i == 0)
        def _():
            s1_ref[...] = jnp.zeros_like(s1_ref)
            s2_ref[...] = jnp.zeros_like(s2_ref)
        s1_ref[...] += lax.dot_general(oh, h1, (((0,), (0,)), ((), ())),
                                       precision=lax.Precision.HIGHEST,
                                       preferred_element_type=jnp.float32)
        s2_ref[...] += lax.dot_general(oh, h1 * h1, (((0,), (0,)), ((), ())),
                                       precision=lax.Precision.HIGHEST,
                                       preferred_element_type=jnp.float32)
    return pl.pallas_call(
        body,
        grid=(GRID,),
        in_specs=[
            pl.BlockSpec((NC, BN, 16), lambda i: (0, i, 0)),
            pl.BlockSpec((BN, F_IN), lambda i: (i, 0)),
            pl.BlockSpec((BN, 8), lambda i: (i, 0)),
            pl.BlockSpec((BN, 1), lambda i: (i, 0)),
            pl.BlockSpec((F_IN, H), lambda i: (0, 0)),
            pl.BlockSpec((1, H), lambda i: (0, 0)),
        ],
        out_specs=[
            pl.BlockSpec((BN, H), lambda i: (i, 0)),
            pl.BlockSpec((B, H), lambda i: (0, 0)),
            pl.BlockSpec((B, H), lambda i: (0, 0)),
        ],
        out_shape=[
            jax.ShapeDtypeStruct((N, H), jnp.float32),
            jax.ShapeDtypeStruct((B, H), jnp.float32),
            jax.ShapeDtypeStruct((B, H), jnp.float32),
        ],
    )(a4, xn, dis8, batch2d, W1T, b1[None, :])


def _gn_apply(h, batch2d, s1, s2, cnt, w, b, ms, dis8=None, relu=False):
    split = dis8 is not None

    def body(*refs):
        if split:
            (h_ref, bt_ref, s1_ref, s2_ref, cnt_ref, w_ref, b_ref, ms_ref,
             dis_ref, o0, o1, o2, o3) = refs
        else:
            (h_ref, bt_ref, s1_ref, s2_ref, cnt_ref, w_ref, b_ref, ms_ref,
             og) = refs
        oh = _onehot(bt_ref[...])
        cntn = jnp.maximum(_bvec(oh, cnt_ref[...]), 1.0)
        s1n = jnp.dot(oh, s1_ref[...], precision=lax.Precision.HIGHEST,
                      preferred_element_type=jnp.float32)
        s2n = jnp.dot(oh, s2_ref[...], precision=lax.Precision.HIGHEST,
                      preferred_element_type=jnp.float32)
        mean = s1n / cntn
        msv = ms_ref[...]
        var = s2n / cntn - (2.0 * msv - msv * msv) * mean * mean
        std = jnp.sqrt(var + 1e-5)
        out = w_ref[...] * (h_ref[...] - mean * msv) / std + b_ref[...]
        if relu:
            out = jnp.maximum(out, 0.0)
        if split:
            out = out * dis_ref[...][:, 0:1]
            o0[...] = out[:, 0:32]
            o1[...] = out[:, 32:64]
            o2[...] = out[:, 64:96]
            o3[...] = out[:, 96:128]
        else:
            og[...] = out

    in_specs = [
        pl.BlockSpec((BN, H), lambda i: (i, 0)),
        pl.BlockSpec((BN, 1), lambda i: (i, 0)),
        pl.BlockSpec((B, H), lambda i: (0, 0)),
        pl.BlockSpec((B, H), lambda i: (0, 0)),
        pl.BlockSpec((1, B), lambda i: (0, 0)),
        pl.BlockSpec((1, H), lambda i: (0, 0)),
        pl.BlockSpec((1, H), lambda i: (0, 0)),
        pl.BlockSpec((1, H), lambda i: (0, 0)),
    ]
    args = [h, batch2d, s1, s2, cnt, w[None, :], b[None, :], ms[None, :]]
    if split:
        in_specs.append(pl.BlockSpec((BN, 8), lambda i: (i, 0)))
        args.append(dis8)
        out_specs = [pl.BlockSpec((BN, 32), lambda i: (i, 0))] * 4
        out_shape = [jax.ShapeDtypeStruct((N, 32), jnp.float32)] * 4
    else:
        out_specs = [pl.BlockSpec((BN, H), lambda i: (i, 0))]
        out_shape = [jax.ShapeDtypeStruct((N, H), jnp.float32)]
    return pl.pallas_call(
        body, grid=(GRID,), in_specs=in_specs,
        out_specs=out_specs, out_shape=out_shape,
    )(*args)


def _conv2(aggs, hps, dis8, batch2d, W2T, b2):
    def body(a0, a1, a2, a3, p0, p1, p2, p3, dis_ref, bt_ref, w_ref, b_ref,
             h3_ref, s1_ref, s2_ref):
        i = pl.program_id(0)
        dis = dis_ref[...][:, 0:1]
        parts = []
        for a_ref, p_ref in ((a0, p0), (a1, p1), (a2, p2), (a3, p3)):
            a = a_ref[...]
            parts.append(dis * (a[0] + a[1] + p_ref[...]))
        A = jnp.concatenate(parts, axis=1)
        h3 = jnp.dot(A, w_ref[...],
                     preferred_element_type=jnp.float32) + b_ref[...]
        h3_ref[...] = h3
        oh = _onehot(bt_ref[...])
        @pl.when(i == 0)
        def _():
            s1_ref[...] = jnp.zeros_like(s1_ref)
            s2_ref[...] = jnp.zeros_like(s2_ref)
        s1_ref[...] += lax.dot_general(oh, h3, (((0,), (0,)), ((), ())),
                                       precision=lax.Precision.HIGHEST,
                                       preferred_element_type=jnp.float32)
        s2_ref[...] += lax.dot_general(oh, h3 * h3, (((0,), (0,)), ((), ())),
                                       precision=lax.Precision.HIGHEST,
                                       preferred_element_type=jnp.float32)
    return pl.pallas_call(
        body,
        grid=(GRID,),
        in_specs=(
            [pl.BlockSpec((NC, BN, 32), lambda i: (0, i, 0))] * 4
            + [pl.BlockSpec((BN, 32), lambda i: (i, 0))] * 4
            + [pl.BlockSpec((BN, 8), lambda i: (i, 0)),
               pl.BlockSpec((BN, 1), lambda i: (i, 0)),
               pl.BlockSpec((H, H), lambda i: (0, 0)),
               pl.BlockSpec((1, H), lambda i: (0, 0))]),
        out_specs=[
            pl.BlockSpec((BN, H), lambda i: (i, 0)),
            pl.BlockSpec((B, H), lambda i: (0, 0)),
            pl.BlockSpec((B, H), lambda i: (0, 0)),
        ],
        out_shape=[
            jax.ShapeDtypeStruct((N, H), jnp.float32),
            jax.ShapeDtypeStruct((B, H), jnp.float32),
            jax.ShapeDtypeStruct((B, H), jnp.float32),
        ],
    )(*aggs, *hps, dis8, batch2d, W2T, b2[None, :])


def _lstm_consts(bih, bhh):
    g = bih + bhh
    i = jax.nn.sigmoid(g[:, 0:H])
    f = jax.nn.sigmoid(g[:, H:2 * H])
    gg = jnp.tanh(g[:, 2 * H:3 * H])
    o = jax.nn.sigmoid(g[:, 3 * H:4 * H])
    c1 = f * 0.0 + i * gg
    q1 = o * jnp.tanh(c1)
    return q1, c1


def _e_pass(hg, batch2d, q, is_table):
    def body(hg_ref, bt_ref, q_ref, e_ref, m_ref):
        i = pl.program_id(0)
        if is_table:
            oh = _onehot(bt_ref[...])
            qn = jnp.dot(oh, q_ref[...], precision=lax.Precision.HIGHEST,
                         preferred_element_type=jnp.float32)
        else:
            qn = q_ref[...]
        e = jnp.sum(hg_ref[...] * qn, axis=1, keepdims=True)
        e_ref[...] = jnp.broadcast_to(e, e_ref.shape)
        bm = jnp.max(e)
        @pl.when(i == 0)
        def _():
            m_ref[...] = jnp.full_like(m_ref, -jnp.inf)
        m_ref[...] = jnp.maximum(m_ref[...], bm)
    return pl.pallas_call(
        body,
        grid=(GRID,),
        in_specs=[
            pl.BlockSpec((BN, H), lambda i: (i, 0)),
            pl.BlockSpec((BN, 1), lambda i: (i, 0)),
            pl.BlockSpec((B if is_table else 1, H), lambda i: (0, 0)),
        ],
        out_specs=[
            pl.BlockSpec((BN, 8), lambda i: (i, 0)),
            pl.BlockSpec((1, 8), lambda i: (0, 0)),
        ],
        out_shape=[
            jax.ShapeDtypeStruct((N, 8), jnp.float32),
            jax.ShapeDtypeStruct((1, 8), jnp.float32),
        ],
    )(hg, batch2d, q)


def _den_pass(e8, m8, batch2d):
    def body(e_ref, m_ref, bt_ref, ex_ref, den_ref):
        i = pl.program_id(0)
        ex = jnp.exp(e_ref[...][:, 0:1] - m_ref[...][0:1, 0:1])
        ex_ref[...] = jnp.broadcast_to(ex, ex_ref.shape)
        oh = _onehot(bt_ref[...])
        @pl.when(i == 0)
        def _():
            den_ref[...] = jnp.zeros_like(den_ref)
        den_ref[...] += jnp.sum(oh * ex, axis=0, keepdims=True)
    return pl.pallas_call(
        body,
        grid=(GRID,),
        in_specs=[
            pl.BlockSpec((BN, 8), lambda i: (i, 0)),
            pl.BlockSpec((1, 8), lambda i: (0, 0)),
            pl.BlockSpec((BN, 1), lambda i: (i, 0)),
        ],
        out_specs=[
            pl.BlockSpec((BN, 8), lambda i: (i, 0)),
            pl.BlockSpec((1, B), lambda i: (0, 0)),
        ],
        out_shape=[
            jax.ShapeDtypeStruct((N, 8), jnp.float32),
            jax.ShapeDtypeStruct((1, B), jnp.float32),
        ],
    )(e8, m8, batch2d)


def _r_pass(hg, ex8, den, batch2d):
    def body(hg_ref, ex_ref, den_ref, bt_ref, r_ref):
        i = pl.program_id(0)
        oh = _onehot(bt_ref[...])
        dn = _bvec(oh, den_ref[...]) + 1e-16
        a = ex_ref[...][:, 0:1] / dn
        @pl.when(i == 0)
        def _():
            r_ref[...] = jnp.zeros_like(r_ref)
        r_ref[...] += lax.dot_general(oh, hg_ref[...] * a,
                                      (((0,), (0,)), ((), ())),
                                      precision=lax.Precision.HIGHEST,
                                      preferred_element_type=jnp.float32)
    return pl.pallas_call(
        body,
        grid=(GRID,),
        in_specs=[
            pl.BlockSpec((BN, H), lambda i: (i, 0)),
            pl.BlockSpec((BN, 8), lambda i: (i, 0)),
            pl.BlockSpec((1, B), lambda i: (0, 0)),
            pl.BlockSpec((BN, 1), lambda i: (i, 0)),
        ],
        out_specs=[pl.BlockSpec((B, H), lambda i: (0, 0))],
        out_shape=[jax.ShapeDtypeStruct((B, H), jnp.float32)],
    )(hg, ex8, den, batch2d)


def _lstm2(r1, WihT, WhhT, bih, bhh):
    def body(r_ref, wih_ref, whh_ref, bih_ref, bhh_ref, q2_ref):
        q1, c1 = _lstm_consts(bih_ref[...], bhh_ref[...])
        q1b = jnp.broadcast_to(q1, (B, H))
        qs1 = jnp.concatenate([q1b, r_ref[...]], axis=1)
        gates = (jnp.dot(qs1, wih_ref[...],
                         preferred_element_type=jnp.float32) + bih_ref[...]
                 + jnp.dot(q1b, whh_ref[...],
                           preferred_element_type=jnp.float32) + bhh_ref[...])
        i = jax.nn.sigmoid(gates[:, 0:H])
        f = jax.nn.sigmoid(gates[:, H:2 * H])
        g = jnp.tanh(gates[:, 2 * H:3 * H])
        o = jax.nn.sigmoid(gates[:, 3 * H:4 * H])
        c2 = f * c1 + i * g
        q2_ref[...] = o * jnp.tanh(c2)
    return pl.pallas_call(
        body,
        out_shape=jax.ShapeDtypeStruct((B, H), jnp.float32),
    )(r1, WihT, WhhT, bih, bhh)


def _head(q2, r2, lin1T, b1, lin2_W, b2):
    def body(q_ref, r_ref, w1_ref, b1_ref, w2_ref, b2_ref, o_ref):
        qs = jnp.concatenate([q_ref[...], r_ref[...]], axis=1)
        h = jnp.maximum(
            jnp.dot(qs, w1_ref[...], preferred_element_type=jnp.float32)
            + b1_ref[...], 0.0)
        o_ref[...] = (jnp.sum(h * w2_ref[...], axis=1, keepdims=True)
                      + b2_ref[...])
    return pl.pallas_call(
        body,
        out_shape=jax.ShapeDtypeStruct((B, 1), jnp.float32),
    )(q2, r2, lin1T, b1[None, :], lin2_W, b2[None, :])


def kernel(x, gn0_w, gn0_b, gn0_ms, conv1_W, conv1_b, gn1_w, gn1_b, gn1_ms,
           conv2_W, conv2_b, gn2_w, gn2_b, gn2_ms,
           lstm_Wih, lstm_Whh, lstm_bih, lstm_bhh,
           lin1_W, lin1_b, lin2_W, lin2_b, edge_index, batch):
    npad = E_PAD - E
    src1d = jnp.concatenate([edge_index[0], jnp.zeros((npad,), jnp.int32)])
    dst1d = jnp.concatenate(
        [edge_index[1], N + (jnp.arange(npad, dtype=jnp.int32) % WIN)])
    batch2d = batch.astype(jnp.int32).reshape(N, 1)
    zrows16 = jnp.zeros((ZCH, 16), jnp.float32)
    zrows32 = jnp.zeros((ZCH, 32), jnp.float32)
    ones_rows = jnp.concatenate(
        [jnp.ones((CHW * WIN, 1), jnp.float32),
         jnp.zeros((CHW * WIN, 15), jnp.float32)], axis=1)

    dcop = _sc_deg(dst1d, ones_rows, zrows16)
    s1x, s2x = _gn0_stats(x)
    hp4, xn, dis8, cnt = _prep(x, s1x, s2x, dcop, batch2d, gn0_w, gn0_b,
                               gn0_ms)
    a4 = _sc_edge_agg(16, hp4, src1d, dst1d, zrows16)
    h1, s1a, s2a = _conv1(a4, xn, dis8, batch2d, conv1_W.T, conv1_b)
    hps = _gn_apply(h1, batch2d, s1a, s2a, cnt, gn1_w, gn1_b, gn1_ms,
                    dis8=dis8, relu=True)
    aggs = [_sc_edge_agg(32, hp_c, src1d, dst1d, zrows32) for hp_c in hps]
    h3, s1b, s2b = _conv2(aggs, hps, dis8, batch2d, conv2_W.T, conv2_b)
    (hg,) = _gn_apply(h3, batch2d, s1b, s2b, cnt, gn2_w, gn2_b, gn2_ms)

    bih2 = lstm_bih[None, :]
    bhh2 = lstm_bhh[None, :]
    q1, _ = _lstm_consts(bih2, bhh2)
    e1, m1 = _e_pass(hg, batch2d, q1, is_table=False)
    ex1, den1 = _den_pass(e1, m1, batch2d)
    r1 = _r_pass(hg, ex1, den1, batch2d)[0]
    q2 = _lstm2(r1, lstm_Wih.T, lstm_Whh.T, bih2, bhh2)
    e2, m2 = _e_pass(hg, batch2d, q2, is_table=True)
    ex2, den2 = _den_pass(e2, m2, batch2d)
    r2 = _r_pass(hg, ex2, den2, batch2d)[0]
    return _head(q2, r2, lin1_W.T, lin1_b, lin2_W, lin2_b)


# speedup vs baseline: 10.4520x; 1.1250x over previous
import functools

import jax
import jax.numpy as jnp
from jax import lax
from jax.experimental import pallas as pl
from jax.experimental.pallas import tpu as pltpu
from jax.experimental.pallas import tpu_sc as plsc

N = 50000
E = 800000
F_IN = 4
H = 128
B = 64

NC = 2
NS = 16
WIN = 128
WPT = 200
EPT = WPT * WIN
E_PAD = NC * NS * EPT
EROWS = E_PAD // WIN
N_PAD = 50176
RPS = N_PAD // NS
ZCH = 112
CHW = 4
CHE = 256
NCHK = EPT // CHE

BN = 1000
GRID = N // BN

_MESH = plsc.VectorSubcoreMesh(core_axis_name="c", subcore_axis_name="s")


def _sc_deg(dst2d, ones_rows, zrows):
    @functools.partial(
        pl.kernel, mesh=_MESH,
        out_type=jax.ShapeDtypeStruct((NC, N_PAD, 16), jnp.float32),
        scratch_types=[
            pltpu.VMEM((CHW * WIN,), jnp.int32),
            pltpu.VMEM((CHW * WIN, 16), jnp.float32),
            pltpu.VMEM((ZCH, 16), jnp.float32),
            pltpu.VMEM_SHARED((N_PAD, 16), jnp.float32),
        ],
        compiler_params=pltpu.CompilerParams(use_tc_tiling_on_sc=False),
    )
    def k(dst_hbm, ones_hbm, z_hbm, out, dst_v, ones_v, z_v, acc):
        c = lax.axis_index("c")
        s = lax.axis_index("s")
        pltpu.sync_copy(z_hbm, z_v)
        def zbody(j, carry):
            pltpu.sync_copy(z_v, acc.at[pl.ds(s * RPS + j * ZCH, ZCH)])
            return carry
        lax.fori_loop(0, RPS // ZCH, zbody, 0)
        pltpu.sync_copy(ones_hbm, ones_v)
        e0 = (c * NS + s) * EPT
        plsc.subcore_barrier()
        def chunk(g, carry):
            pltpu.sync_copy(dst_hbm.at[pl.ds(e0 + g * (CHW * WIN),
                                             CHW * WIN)], dst_v)
            pltpu.sync_copy(ones_v, acc.at[dst_v], add=True)
            return carry
        lax.fori_loop(0, WPT // CHW, chunk, 0)
        plsc.subcore_barrier()
        pltpu.sync_copy(acc.at[pl.ds(s * RPS, RPS)],
                        out.at[c, pl.ds(s * RPS, RPS)])
    return k(dst2d, ones_rows, zrows)


def _sc_edge_agg(ncols, hp, src1d, dst1d, zrows):
    @functools.partial(
        pl.kernel, mesh=_MESH,
        out_type=jax.ShapeDtypeStruct((NC, N_PAD, ncols), jnp.float32),
        scratch_types=[
            pltpu.VMEM((CHE,), jnp.int32),
            pltpu.VMEM((CHE,), jnp.int32),
            pltpu.VMEM((CHE, ncols), jnp.float32),
            pltpu.VMEM((CHE,), jnp.int32),
            pltpu.VMEM((CHE,), jnp.int32),
            pltpu.VMEM((CHE, ncols), jnp.float32),
            pltpu.VMEM((ZCH, ncols), jnp.float32),
            pltpu.VMEM_SHARED((N_PAD, ncols), jnp.float32),
            pltpu.SemaphoreType.DMA,
            pltpu.SemaphoreType.DMA,
        ],
        compiler_params=pltpu.CompilerParams(use_tc_tiling_on_sc=False),
    )
    def k(hp_hbm, src_hbm, dst_hbm, z_hbm, out, s0, d0, r0, s1, d1, r1,
          z_v, acc, m0, m1):
        c = lax.axis_index("c")
        s = lax.axis_index("s")
        bufs = ((s0, d0, r0, m0), (s1, d1, r1, m1))
        pltpu.sync_copy(z_hbm, z_v)
        def zbody(j, carry):
            pltpu.sync_copy(z_v, acc.at[pl.ds(s * RPS + j * ZCH, ZCH)])
            return carry
        lax.fori_loop(0, RPS // ZCH, zbody, 0)
        e0 = (c * NS + s) * EPT
        plsc.subcore_barrier()

        def stage_fire(g, b):
            sv, dv, rv, sem = bufs[b]
            pltpu.sync_copy(src_hbm.at[pl.ds(e0 + g * CHE, CHE)], sv)
            pltpu.sync_copy(dst_hbm.at[pl.ds(e0 + g * CHE, CHE)], dv)
            pltpu.async_copy(hp_hbm.at[sv], rv, sem)

        stage_fire(0, 0)
        stage_fire(1, 1)

        def outer(gg, carry):
            for b in range(2):
                g = gg * 2 + b
                sv, dv, rv, sem = bufs[b]
                pltpu.make_async_copy(hp_hbm.at[pl.ds(0, CHE)], rv,
                                      sem).wait()
                pltpu.sync_copy(rv, acc.at[dv], add=True)
                @pl.when(g + 2 < NCHK)
                def _():
                    stage_fire(g + 2, b)
            return carry
        lax.fori_loop(0, NCHK // 2, outer, 0)
        plsc.subcore_barrier()
        pltpu.sync_copy(acc.at[pl.ds(s * RPS, RPS)],
                        out.at[c, pl.ds(s * RPS, RPS)])
    return k(hp, src1d, dst1d, zrows)


def _onehot(batch_blk):
    iota = lax.broadcasted_iota(jnp.int32, (1, B), 1)
    return (batch_blk == iota).astype(jnp.float32)


def _bvec(onehot, table_row):
    return jnp.sum(onehot * table_row, axis=1, keepdims=True)


def _gn0_stats(x):
    def body(x_ref, s1_ref, s2_ref):
        i = pl.program_id(0)
        @pl.when(i == 0)
        def _():
            s1_ref[...] = jnp.zeros_like(s1_ref)
            s2_ref[...] = jnp.zeros_like(s2_ref)
        xb = x_ref[...]
        s1_ref[...] += jnp.sum(xb, axis=0, keepdims=True)
        s2_ref[...] += jnp.sum(xb * xb, axis=0, keepdims=True)
    return pl.pallas_call(
        body,
        grid=(GRID,),
        in_specs=[pl.BlockSpec((BN, F_IN), lambda i: (i, 0))],
        out_specs=[pl.BlockSpec((1, F_IN), lambda i: (0, 0)),
                   pl.BlockSpec((1, F_IN), lambda i: (0, 0))],
        out_shape=[jax.ShapeDtypeStruct((1, F_IN), jnp.float32)] * 2,
    )(x)


def _prep(x, s1, s2, dcop, batch2d, w, b, ms):
    def body(x_ref, s1_ref, s2_ref, d_ref, bt_ref, w_ref, b_ref, ms_ref,
             hp4_ref, xn_ref, dis_ref, cnt_ref):
        i = pl.program_id(0)
        mean = s1_ref[...] / float(N)
        msv = ms_ref[...]
        var = (s2_ref[...] / float(N)
               - (2.0 * msv - msv * msv) * mean * mean)
        std = jnp.sqrt(var + 1e-5)
        xb = x_ref[...]
        xn = w_ref[...] * (xb - mean * msv) / std + b_ref[...]
        d = d_ref[...]
        deg = d[0, :, 0:1] + d[1, :, 0:1] + 1.0
        dis = lax.rsqrt(deg)
        xn_ref[...] = xn
        dis_ref[...] = jnp.broadcast_to(dis, dis_ref.shape)
        hp4_ref[...] = jnp.concatenate(
            [xn * dis, jnp.zeros((xn.shape[0], 16 - F_IN), jnp.float32)],
            axis=1)
        oh = _onehot(bt_ref[...])
        @pl.when(i == 0)
        def _():
            cnt_ref[...] = jnp.zeros_like(cnt_ref)
        cnt_ref[...] += jnp.sum(oh, axis=0, keepdims=True)
    return pl.pallas_call(
        body,
        grid=(GRID,),
        in_specs=[
            pl.BlockSpec((BN, F_IN), lambda i: (i, 0)),
            pl.BlockSpec((1, F_IN), lambda i: (0, 0)),
            pl.BlockSpec((1, F_IN), lambda i: (0, 0)),
            pl.BlockSpec((NC, BN, 16), lambda i: (0, i, 0)),
            pl.BlockSpec((BN, 1), lambda i: (i, 0)),
            pl.BlockSpec((1, F_IN), lambda i: (0, 0)),
            pl.BlockSpec((1, F_IN), lambda i: (0, 0)),
            pl.BlockSpec((1, F_IN), lambda i: (0, 0)),
        ],
        out_specs=[
            pl.BlockSpec((BN, 16), lambda i: (i, 0)),
            pl.BlockSpec((BN, F_IN), lambda i: (i, 0)),
            pl.BlockSpec((BN, 8), lambda i: (i, 0)),
            pl.BlockSpec((1, B), lambda i: (0, 0)),
        ],
        out_shape=[
            jax.ShapeDtypeStruct((N, 16), jnp.float32),
            jax.ShapeDtypeStruct((N, F_IN), jnp.float32),
            jax.ShapeDtypeStruct((N, 8), jnp.float32),
            jax.ShapeDtypeStruct((1, B), jnp.float32),
        ],
    )(x, s1, s2, dcop, batch2d, w[None, :], b[None, :], ms[None, :])


def _conv1(a4, xn, dis8, batch2d, W1T, b1):
    def body(a_ref, xn_ref, dis_ref, bt_ref, w_ref, b_ref,
             h1_ref, s1_ref, s2_ref):
        i = pl.program_id(0)
        a = a_ref[...]
        dis = dis_ref[...][:, 0:1]
        agg = (a[0, :, 0:F_IN] + a[1, :, 0:F_IN])
        A4 = dis * agg + (dis * dis) * xn_ref[...]
        h1 = jnp.dot(A4, w_ref[...],
                     preferred_element_type=jnp.float32) + b_ref[...]
        h1_ref[...] = h1
        oh = _onehot(bt_ref[...])
        @pl.when(i == 0)
        def _():
            s1_ref[...] = jnp.zeros_like(s1_ref)
            s2_ref[...] = jnp.zeros_like(s2_ref)
        s1_ref[...] += lax.dot_general(oh, h1, (((0,), (0,)), ((), ())),
                                       precision=lax.Precision.HIGHEST,
                                       preferred_element_type=jnp.float32)
        s2_ref[...] += lax.dot_general(oh, h1 * h1, (((0,), (0,)), ((), ())),
                                       precision=lax.Precision.HIGHEST,
                                       preferred_element_type=jnp.float32)
    return pl.pallas_call(
        body,
        grid=(GRID,),
        in_specs=[
            pl.BlockSpec((NC, BN, 16), lambda i: (0, i, 0)),
            pl.BlockSpec((BN, F_IN), lambda i: (i, 0)),
            pl.BlockSpec((BN, 8), lambda i: (i, 0)),
            pl.BlockSpec((BN, 1), lambda i: (i, 0)),
            pl.BlockSpec((F_IN, H), lambda i: (0, 0)),
            pl.BlockSpec((1, H), lambda i: (0, 0)),
        ],
        out_specs=[
            pl.BlockSpec((BN, H), lambda i: (i, 0)),
            pl.BlockSpec((B, H), lambda i: (0, 0)),
            pl.BlockSpec((B, H), lambda i: (0, 0)),
        ],
        out_shape=[
            jax.ShapeDtypeStruct((N, H), jnp.float32),
            jax.ShapeDtypeStruct((B, H), jnp.float32),
            jax.ShapeDtypeStruct((B, H), jnp.float32),
        ],
    )(a4, xn, dis8, batch2d, W1T, b1[None, :])


def _gn_apply(h, batch2d, s1, s2, cnt, w, b, ms, dis8=None, relu=False):
    split = dis8 is not None

    def body(*refs):
        if split:
            (h_ref, bt_ref, s1_ref, s2_ref, cnt_ref, w_ref, b_ref, ms_ref,
             dis_ref, o0, o1, o2, o3) = refs
        else:
            (h_ref, bt_ref, s1_ref, s2_ref, cnt_ref, w_ref, b_ref, ms_ref,
             og) = refs
        oh = _onehot(bt_ref[...])
        cntn = jnp.maximum(_bvec(oh, cnt_ref[...]), 1.0)
        s1n = jnp.dot(oh, s1_ref[...], precision=lax.Precision.HIGHEST,
                      preferred_element_type=jnp.float32)
        s2n = jnp.dot(oh, s2_ref[...], precision=lax.Precision.HIGHEST,
                      preferred_element_type=jnp.float32)
        mean = s1n / cntn
        msv = ms_ref[...]
        var = s2n / cntn - (2.0 * msv - msv * msv) * mean * mean
        std = jnp.sqrt(var + 1e-5)
        out = w_ref[...] * (h_ref[...] - mean * msv) / std + b_ref[...]
        if relu:
            out = jnp.maximum(out, 0.0)
        if split:
            out = out * dis_ref[...][:, 0:1]
            o0[...] = out[:, 0:32]
            o1[...] = out[:, 32:64]
            o2[...] = out[:, 64:96]
            o3[...] = out[:, 96:128]
        else:
            og[...] = out

    in_specs = [
        pl.BlockSpec((BN, H), lambda i: (i, 0)),
        pl.BlockSpec((BN, 1), lambda i: (i, 0)),
        pl.BlockSpec((B, H), lambda i: (0, 0)),
        pl.BlockSpec((B, H), lambda i: (0, 0)),
        pl.BlockSpec((1, B), lambda i: (0, 0)),
        pl.BlockSpec((1, H), lambda i: (0, 0)),
        pl.BlockSpec((1, H), lambda i: (0, 0)),
        pl.BlockSpec((1, H), lambda i: (0, 0)),
    ]
    args = [h, batch2d, s1, s2, cnt, w[None, :], b[None, :], ms[None, :]]
    if split:
        in_specs.append(pl.BlockSpec((BN, 8), lambda i: (i, 0)))
        args.append(dis8)
        out_specs = [pl.BlockSpec((BN, 32), lambda i: (i, 0))] * 4
        out_shape = [jax.ShapeDtypeStruct((N, 32), jnp.float32)] * 4
    else:
        out_specs = [pl.BlockSpec((BN, H), lambda i: (i, 0))]
        out_shape = [jax.ShapeDtypeStruct((N, H), jnp.float32)]
    return pl.pallas_call(
        body, grid=(GRID,), in_specs=in_specs,
        out_specs=out_specs, out_shape=out_shape,
    )(*args)


def _conv2(aggs, hps, dis8, batch2d, W2T, b2):
    def body(a0, a1, a2, a3, p0, p1, p2, p3, dis_ref, bt_ref, w_ref, b_ref,
             h3_ref, s1_ref, s2_ref):
        i = pl.program_id(0)
        dis = dis_ref[...][:, 0:1]
        parts = []
        for a_ref, p_ref in ((a0, p0), (a1, p1), (a2, p2), (a3, p3)):
            a = a_ref[...]
            parts.append(dis * (a[0] + a[1] + p_ref[...]))
        A = jnp.concatenate(parts, axis=1)
        h3 = jnp.dot(A, w_ref[...],
                     preferred_element_type=jnp.float32) + b_ref[...]
        h3_ref[...] = h3
        oh = _onehot(bt_ref[...])
        @pl.when(i == 0)
        def _():
            s1_ref[...] = jnp.zeros_like(s1_ref)
            s2_ref[...] = jnp.zeros_like(s2_ref)
        s1_ref[...] += lax.dot_general(oh, h3, (((0,), (0,)), ((), ())),
                                       precision=lax.Precision.HIGHEST,
                                       preferred_element_type=jnp.float32)
        s2_ref[...] += lax.dot_general(oh, h3 * h3, (((0,), (0,)), ((), ())),
                                       precision=lax.Precision.HIGHEST,
                                       preferred_element_type=jnp.float32)
    return pl.pallas_call(
        body,
        grid=(GRID,),
        in_specs=(
            [pl.BlockSpec((NC, BN, 32), lambda i: (0, i, 0))] * 4
            + [pl.BlockSpec((BN, 32), lambda i: (i, 0))] * 4
            + [pl.BlockSpec((BN, 8), lambda i: (i, 0)),
               pl.BlockSpec((BN, 1), lambda i: (i, 0)),
               pl.BlockSpec((H, H), lambda i: (0, 0)),
               pl.BlockSpec((1, H), lambda i: (0, 0))]),
        out_specs=[
            pl.BlockSpec((BN, H), lambda i: (i, 0)),
            pl.BlockSpec((B, H), lambda i: (0, 0)),
            pl.BlockSpec((B, H), lambda i: (0, 0)),
        ],
        out_shape=[
            jax.ShapeDtypeStruct((N, H), jnp.float32),
            jax.ShapeDtypeStruct((B, H), jnp.float32),
            jax.ShapeDtypeStruct((B, H), jnp.float32),
        ],
    )(*aggs, *hps, dis8, batch2d, W2T, b2[None, :])


def _lstm_consts(bih, bhh):
    g = bih + bhh
    i = jax.nn.sigmoid(g[:, 0:H])
    f = jax.nn.sigmoid(g[:, H:2 * H])
    gg = jnp.tanh(g[:, 2 * H:3 * H])
    o = jax.nn.sigmoid(g[:, 3 * H:4 * H])
    c1 = f * 0.0 + i * gg
    q1 = o * jnp.tanh(c1)
    return q1, c1


def _e_pass(hg, batch2d, q, is_table):
    def body(hg_ref, bt_ref, q_ref, e_ref, m_ref):
        i = pl.program_id(0)
        if is_table:
            oh = _onehot(bt_ref[...])
            qn = jnp.dot(oh, q_ref[...], precision=lax.Precision.HIGHEST,
                         preferred_element_type=jnp.float32)
        else:
            qn = q_ref[...]
        e = jnp.sum(hg_ref[...] * qn, axis=1, keepdims=True)
        e_ref[...] = jnp.broadcast_to(e, e_ref.shape)
        bm = jnp.max(e)
        @pl.when(i == 0)
        def _():
            m_ref[...] = jnp.full_like(m_ref, -jnp.inf)
        m_ref[...] = jnp.maximum(m_ref[...], bm)
    return pl.pallas_call(
        body,
        grid=(GRID,),
        in_specs=[
            pl.BlockSpec((BN, H), lambda i: (i, 0)),
            pl.BlockSpec((BN, 1), lambda i: (i, 0)),
            pl.BlockSpec((B if is_table else 1, H), lambda i: (0, 0)),
        ],
        out_specs=[
            pl.BlockSpec((BN, 8), lambda i: (i, 0)),
            pl.BlockSpec((1, 8), lambda i: (0, 0)),
        ],
        out_shape=[
            jax.ShapeDtypeStruct((N, 8), jnp.float32),
            jax.ShapeDtypeStruct((1, 8), jnp.float32),
        ],
    )(hg, batch2d, q)


def _den_pass(e8, m8, batch2d):
    def body(e_ref, m_ref, bt_ref, ex_ref, den_ref):
        i = pl.program_id(0)
        ex = jnp.exp(e_ref[...][:, 0:1] - m_ref[...][0:1, 0:1])
        ex_ref[...] = jnp.broadcast_to(ex, ex_ref.shape)
        oh = _onehot(bt_ref[...])
        @pl.when(i == 0)
        def _():
            den_ref[...] = jnp.zeros_like(den_ref)
        den_ref[...] += jnp.sum(oh * ex, axis=0, keepdims=True)
    return pl.pallas_call(
        body,
        grid=(GRID,),
        in_specs=[
            pl.BlockSpec((BN, 8), lambda i: (i, 0)),
            pl.BlockSpec((1, 8), lambda i: (0, 0)),
            pl.BlockSpec((BN, 1), lambda i: (i, 0)),
        ],
        out_specs=[
            pl.BlockSpec((BN, 8), lambda i: (i, 0)),
            pl.BlockSpec((1, B), lambda i: (0, 0)),
        ],
        out_shape=[
            jax.ShapeDtypeStruct((N, 8), jnp.float32),
            jax.ShapeDtypeStruct((1, B), jnp.float32),
        ],
    )(e8, m8, batch2d)


def _r_pass(hg, ex8, den, batch2d):
    def body(hg_ref, ex_ref, den_ref, bt_ref, r_ref):
        i = pl.program_id(0)
        oh = _onehot(bt_ref[...])
        dn = _bvec(oh, den_ref[...]) + 1e-16
        a = ex_ref[...][:, 0:1] / dn
        @pl.when(i == 0)
        def _():
            r_ref[...] = jnp.zeros_like(r_ref)
        r_ref[...] += lax.dot_general(oh, hg_ref[...] * a,
                                      (((0,), (0,)), ((), ())),
                                      precision=lax.Precision.HIGHEST,
                                      preferred_element_type=jnp.float32)
    return pl.pallas_call(
        body,
        grid=(GRID,),
        in_specs=[
            pl.BlockSpec((BN, H), lambda i: (i, 0)),
            pl.BlockSpec((BN, 8), lambda i: (i, 0)),
            pl.BlockSpec((1, B), lambda i: (0, 0)),
            pl.BlockSpec((BN, 1), lambda i: (i, 0)),
        ],
        out_specs=[pl.BlockSpec((B, H), lambda i: (0, 0))],
        out_shape=[jax.ShapeDtypeStruct((B, H), jnp.float32)],
    )(hg, ex8, den, batch2d)


def _lstm2(r1, WihT, WhhT, bih, bhh):
    def body(r_ref, wih_ref, whh_ref, bih_ref, bhh_ref, q2_ref):
        q1, c1 = _lstm_consts(bih_ref[...], bhh_ref[...])
        q1b = jnp.broadcast_to(q1, (B, H))
        qs1 = jnp.concatenate([q1b, r_ref[...]], axis=1)
        gates = (jnp.dot(qs1, wih_ref[...],
                         preferred_element_type=jnp.float32) + bih_ref[...]
                 + jnp.dot(q1b, whh_ref[...],
                           preferred_element_type=jnp.float32) + bhh_ref[...])
        i = jax.nn.sigmoid(gates[:, 0:H])
        f = jax.nn.sigmoid(gates[:, H:2 * H])
        g = jnp.tanh(gates[:, 2 * H:3 * H])
        o = jax.nn.sigmoid(gates[:, 3 * H:4 * H])
        c2 = f * c1 + i * g
        q2_ref[...] = o * jnp.tanh(c2)
    return pl.pallas_call(
        body,
        out_shape=jax.ShapeDtypeStruct((B, H), jnp.float32),
    )(r1, WihT, WhhT, bih, bhh)


def _head(q2, r2, lin1T, b1, lin2_W, b2):
    def body(q_ref, r_ref, w1_ref, b1_ref, w2_ref, b2_ref, o_ref):
        qs = jnp.concatenate([q_ref[...], r_ref[...]], axis=1)
        h = jnp.maximum(
            jnp.dot(qs, w1_ref[...], preferred_element_type=jnp.float32)
            + b1_ref[...], 0.0)
        o_ref[...] = (jnp.sum(h * w2_ref[...], axis=1, keepdims=True)
                      + b2_ref[...])
    return pl.pallas_call(
        body,
        out_shape=jax.ShapeDtypeStruct((B, 1), jnp.float32),
    )(q2, r2, lin1T, b1[None, :], lin2_W, b2[None, :])


def kernel(x, gn0_w, gn0_b, gn0_ms, conv1_W, conv1_b, gn1_w, gn1_b, gn1_ms,
           conv2_W, conv2_b, gn2_w, gn2_b, gn2_ms,
           lstm_Wih, lstm_Whh, lstm_bih, lstm_bhh,
           lin1_W, lin1_b, lin2_W, lin2_b, edge_index, batch):
    npad = E_PAD - E
    src1d = jnp.concatenate([edge_index[0], jnp.zeros((npad,), jnp.int32)])
    dst1d = jnp.concatenate(
        [edge_index[1], N + (jnp.arange(npad, dtype=jnp.int32) % WIN)])
    batch2d = batch.astype(jnp.int32).reshape(N, 1)
    zrows16 = jnp.zeros((ZCH, 16), jnp.float32)
    zrows32 = jnp.zeros((ZCH, 32), jnp.float32)
    ones_rows = jnp.concatenate(
        [jnp.ones((CHW * WIN, 1), jnp.float32),
         jnp.zeros((CHW * WIN, 15), jnp.float32)], axis=1)

    dcop = _sc_deg(dst1d, ones_rows, zrows16)
    s1x, s2x = _gn0_stats(x)
    hp4, xn, dis8, cnt = _prep(x, s1x, s2x, dcop, batch2d, gn0_w, gn0_b,
                               gn0_ms)
    a4 = _sc_edge_agg(16, hp4, src1d, dst1d, zrows16)
    h1, s1a, s2a = _conv1(a4, xn, dis8, batch2d, conv1_W.T, conv1_b)
    hps = _gn_apply(h1, batch2d, s1a, s2a, cnt, gn1_w, gn1_b, gn1_ms,
                    dis8=dis8, relu=True)
    aggs = [_sc_edge_agg(32, hp_c, src1d, dst1d, zrows32) for hp_c in hps]
    h3, s1b, s2b = _conv2(aggs, hps, dis8, batch2d, conv2_W.T, conv2_b)
    (hg,) = _gn_apply(h3, batch2d, s1b, s2b, cnt, gn2_w, gn2_b, gn2_ms)

    bih2 = lstm_bih[None, :]
    bhh2 = lstm_bhh[None, :]
    q1, _ = _lstm_consts(bih2, bhh2)
    e1, m1 = _e_pass(hg, batch2d, q1, is_table=False)
    ex1, den1 = _den_pass(e1, m1, batch2d)
    r1 = _r_pass(hg, ex1, den1, batch2d)[0]
    q2 = _lstm2(r1, lstm_Wih.T, lstm_Whh.T, bih2, bhh2)
    e2, m2 = _e_pass(hg, batch2d, q2, is_table=True)
    ex2, den2 = _den_pass(e2, m2, batch2d)
    r2 = _r_pass(hg, ex2, den2, batch2d)[0]
    return _head(q2, r2, lin1_W.T, lin1_b, lin2_W, lin2_b)
